# baseline (speedup 1.0000x reference)
import functools

import jax
import jax.numpy as jnp
from jax import lax
from jax.experimental import pallas as pl
from jax.experimental.pallas import tpu as pltpu
from jax.experimental.pallas import tpu_sc as plsc

N = 10000
H = 128
E = 320000
NC = 2
NS = 16
NW = NC * NS
K = 112
NCH = 90
EWP = K * NCH
PW = 2 * K + 16
NP = 10240
RPT = NP // NS


NRB = 3
NIB = 6
PEEL = 2
TAIL = 4


def _sc_edge_body(m_hbm, pack_hbm, dst_hbm, z_hbm, out_hbm,
                  pack_v, dst_v, rows_v, agg_sh, psem, dsem, gsem, ssem):
    c = lax.axis_index("c")
    s = lax.axis_index("s")
    wid = s * NC + c
    cbase = wid * NCH

    def idx_fetch(ci, b6):
        pltpu.async_copy(pack_hbm.at[pl.ds((cbase + ci) * PW, PW)],
                         pack_v.at[pl.ds(b6 * PW, PW)], psem.at[b6])
        pltpu.async_copy(dst_hbm.at[pl.ds((cbase + ci) * K, K)],
                         dst_v.at[b6], dsem.at[b6])

    def g_start(ci, b6, b3):
        pltpu.make_async_copy(pack_hbm.at[pl.ds((cbase + ci) * PW, PW)],
                              pack_v.at[pl.ds(b6 * PW, PW)],
                              psem.at[b6]).wait()
        pltpu.async_copy(m_hbm.at[pack_v.at[pl.ds(b6 * PW, K)]],
                         rows_v.at[b3], gsem.at[b3])

    def g_wait(b6, b3):
        pltpu.make_async_copy(m_hbm.at[pack_v.at[pl.ds(b6 * PW, K)]],
                              rows_v.at[b3], gsem.at[b3]).wait()

    def d_wait(ci, b6):
        pltpu.make_async_copy(dst_hbm.at[pl.ds((cbase + ci) * K, K)],
                              dst_v.at[b6], dsem.at[b6]).wait()

    def s_start(b3, b6):
        pltpu.async_copy(rows_v.at[b3], agg_sh.at[dst_v.at[b6]],
                         ssem.at[b3], add=True)

    def s_wait(b3, b6):
        pltpu.make_async_copy(rows_v.at[b3], agg_sh.at[dst_v.at[b6]],
                              ssem.at[b3]).wait()

    def scale(b3, b6):
        def grp(g, _):
            bits = pack_v[pl.ds(b6 * PW + K + g * 16, 16)]
            eag = lax.bitcast_convert_type(bits, jnp.float32)
            for l in range(16):
                eav = jnp.full((16,), eag[l], jnp.float32)
                e = g * 16 + l
                for j in range(H // 16):
                    sl = pl.ds(16 * j, 16)
                    rows_v[b3, e, sl] = rows_v[b3, e, sl] * eav
            return 0

        lax.fori_loop(0, K // 16, grp, 0)

    for ci in range(4):
        idx_fetch(ci, ci)
    g_start(0, 0, 0)
    pltpu.sync_copy(z_hbm, agg_sh.at[pl.ds(s * RPT, RPT)])
    plsc.subcore_barrier()

    def step(ci, b6, b3, first=False):
        g_wait(b6, b3)
        d_wait(ci, b6)
        if not first:
            s_wait((b3 + 1) % NRB, (b6 + 4) % NIB)
        idx_fetch(ci + 4, (b6 + 4) % NIB)
        g_start(ci + 1, (b6 + 1) % NIB, (b3 + 1) % NRB)
        scale(b3, b6)
        s_start(b3, b6)

    for ci in range(PEEL):
        step(ci, ci % NIB, ci % NRB, first=True)

    def sextet(t, carry):
        for k in range(6):
            ci = PEEL + 6 * t + k
            step(ci, (PEEL + k) % NIB, (PEEL + k) % NRB)
        return carry

    lax.fori_loop(0, (NCH - PEEL - TAIL) // 6, sextet, 0)

    for k in range(TAIL):
        ci = NCH - TAIL + k
        b6, b3 = ci % NIB, ci % NRB
        g_wait(b6, b3)
        d_wait(ci, b6)
        s_wait((b3 + 1) % NRB, (b6 + 4) % NIB)
        if ci + 4 < NCH:
            idx_fetch(ci + 4, (b6 + 4) % NIB)
        if ci + 1 < NCH:
            g_start(ci + 1, (b6 + 1) % NIB, (b3 + 1) % NRB)
        scale(b3, b6)
        s_start(b3, b6)

    s_wait((NCH - 2) % NRB, (NCH - 2) % NIB)
    s_wait((NCH - 1) % NRB, (NCH - 1) % NIB)
    plsc.subcore_barrier()

    pltpu.sync_copy(agg_sh.at[pl.ds(s * RPT, RPT)],
                    out_hbm.at[c, pl.ds(s * RPT, RPT)])


@functools.partial(
    pl.kernel,
    out_type=jax.ShapeDtypeStruct((2, NP, H), jnp.float32),
    mesh=plsc.VectorSubcoreMesh(core_axis_name="c", subcore_axis_name="s"),
    scratch_types=[
        pltpu.VMEM((NIB * PW,), jnp.int32),
        pltpu.VMEM((NIB, K), jnp.int32),
        pltpu.VMEM((NRB, K, H), jnp.float32),
        pltpu.VMEM_SHARED((NP, H), jnp.float32),
        pltpu.SemaphoreType.DMA((NIB,)),
        pltpu.SemaphoreType.DMA((NIB,)),
        pltpu.SemaphoreType.DMA((NRB,)),
        pltpu.SemaphoreType.DMA((NRB,)),
    ],
)
def _sc_edge(m_hbm, pack_hbm, dst_hbm, z_hbm, out_hbm,
             pack_v, dst_v, rows_v, agg_sh, psem, dsem, gsem, ssem):
    _sc_edge_body(m_hbm, pack_hbm, dst_hbm, z_hbm, out_hbm,
                  pack_v, dst_v, rows_v, agg_sh, psem, dsem, gsem, ssem)


BN = 1000


def _mm_body(x_ref, w_ref, o_ref):
    o_ref[...] = jnp.dot(x_ref[...], w_ref[...],
                         preferred_element_type=jnp.float32)


def _tc_matmul(x, w):
    return pl.pallas_call(
        _mm_body,
        grid=(N // BN,),
        in_specs=[pl.BlockSpec((BN, H), lambda i: (i, 0)),
                  pl.BlockSpec((H, H), lambda i: (0, 0))],
        out_specs=pl.BlockSpec((BN, H), lambda i: (i, 0)),
        out_shape=jax.ShapeDtypeStruct((N, H), jnp.float32),
    )(x, w)


def _gru_math(p0, p1, h, wihT, whhT, bih, bhh):
    agg = p0 + p1
    gi = jnp.dot(agg, wihT, preferred_element_type=jnp.float32) + bih
    gh = jnp.dot(h, whhT, preferred_element_type=jnp.float32) + bhh
    r = jax.nn.sigmoid(gi[:, :H] + gh[:, :H])
    z = jax.nn.sigmoid(gi[:, H:2 * H] + gh[:, H:2 * H])
    n = jnp.tanh(gi[:, 2 * H:] + r * gh[:, 2 * H:])
    return (1.0 - z) * n + z * h


def _gru_fused_body(p0_ref, p1_ref, h_ref, wihT_ref, whhT_ref, bih_ref,
                    bhh_ref, wn_ref, oh_ref, om_ref):
    hn = _gru_math(p0_ref[0], p1_ref[0], h_ref[...], wihT_ref[...],
                   whhT_ref[...], bih_ref[...], bhh_ref[...])
    oh_ref[...] = hn
    om_ref[...] = jnp.dot(hn, wn_ref[...], preferred_element_type=jnp.float32)


def _gru_final_body(p0_ref, p1_ref, h_ref, wihT_ref, whhT_ref, bih_ref,
                    bhh_ref, oh_ref):
    oh_ref[...] = _gru_math(p0_ref[0], p1_ref[0], h_ref[...], wihT_ref[...],
                            whhT_ref[...], bih_ref[...], bhh_ref[...])


def _blk(r, c):
    return pl.BlockSpec((r, c), lambda i: (i, 0))


def _full(r, c):
    return pl.BlockSpec((r, c), lambda i: (0, 0))


def _pblk(core):
    return pl.BlockSpec((1, BN, H), lambda i, core=core: (core, i, 0))


def _tc_gru_fused(p, h, wihT, whhT, bih, bhh, wn):
    return pl.pallas_call(
        _gru_fused_body,
        grid=(N // BN,),
        in_specs=[_pblk(0), _pblk(1), _blk(BN, H),
                  _full(H, 3 * H), _full(H, 3 * H),
                  _full(1, 3 * H), _full(1, 3 * H), _full(H, H)],
        out_specs=[_blk(BN, H), _blk(BN, H)],
        out_shape=[jax.ShapeDtypeStruct((N, H), jnp.float32),
                   jax.ShapeDtypeStruct((N, H), jnp.float32)],
    )(p, p, h, wihT, whhT, bih, bhh, wn)


def _tc_gru_final(p, h, wihT, whhT, bih, bhh):
    return pl.pallas_call(
        _gru_final_body,
        grid=(N // BN,),
        in_specs=[_pblk(0), _pblk(1), _blk(BN, H),
                  _full(H, 3 * H), _full(H, 3 * H),
                  _full(1, 3 * H), _full(1, 3 * H)],
        out_specs=_blk(BN, H),
        out_shape=jax.ShapeDtypeStruct((N, H), jnp.float32),
    )(p, p, h, wihT, whhT, bih, bhh)


def kernel(x, edge_index, edge_attr, weight, w_ih, w_hh, b_ih, b_hh):
    src = edge_index[0].astype(jnp.int32)
    dst = edge_index[1].astype(jnp.int32)
    ea = edge_attr.astype(jnp.float32)

    pad = EWP - E // NW
    srcp = jnp.concatenate(
        [src.reshape(NW, E // NW),
         jnp.zeros((NW, pad), jnp.int32)], axis=1).reshape(NW, NCH, K)
    dstp = jnp.concatenate(
        [dst.reshape(NW, E // NW),
         jnp.zeros((NW, pad), jnp.int32)], axis=1).reshape(-1)
    eap = jnp.concatenate(
        [lax.bitcast_convert_type(ea, jnp.int32).reshape(NW, E // NW),
         jnp.zeros((NW, pad), jnp.int32)], axis=1).reshape(NW, NCH, K)
    pack = jnp.concatenate(
        [srcp, eap, jnp.zeros((NW, NCH, PW - 2 * K), jnp.int32)],
        axis=2).reshape(-1)

    wihT = w_ih.T
    whhT = w_hh.T
    bih = b_ih.reshape(1, 3 * H)
    bhh = b_hh.reshape(1, 3 * H)
    zeros = jnp.zeros((RPT, H), jnp.float32)

    h = x
    m = _tc_matmul(h, weight[0])
    p = _sc_edge(m, pack, dstp, zeros)
    h, m = _tc_gru_fused(p, h, wihT, whhT, bih, bhh, weight[1])
    p = _sc_edge(m, pack, dstp, zeros)
    h = _tc_gru_final(p, h, wihT, whhT, bih, bhh)
    return h


# speedup vs baseline: 1.3894x; 1.3894x over previous
import functools

import jax
import jax.numpy as jnp
from jax import lax
from jax.experimental import pallas as pl
from jax.experimental.pallas import tpu as pltpu
from jax.experimental.pallas import tpu_sc as plsc

N = 10000
H = 128
E = 320000
NC = 2
NS = 16
NW = NC * NS
EW = E // NW
K = 80
NCH = EW // K
NP = 10240
RPT = NP // NS


NRB = 3
NIB = 6
PEEL = 5


def _sc_edge_body(m_hbm, src_hbm, dst_hbm, ea_hbm, z_hbm, out_hbm,
                  src_v, dst_v, ea_v, rows_v, agg_sh,
                  gsem, srcsem, iesem, ssem):
    c = lax.axis_index("c")
    s = lax.axis_index("s")
    wid = s * NC + c
    base = wid * EW

    def idx_fetch(ci, b6):
        off = base + ci * K
        pltpu.async_copy(src_hbm.at[pl.ds(off, K)], src_v.at[b6],
                         srcsem.at[b6])
        pltpu.async_copy(dst_hbm.at[pl.ds(off, K)], dst_v.at[b6],
                         iesem.at[b6])
        pltpu.async_copy(ea_hbm.at[pl.ds(off, K)],
                         ea_v.at[b6, pl.ds(0, K)], iesem.at[b6])

    def g_start(ci, b6, b3):
        pltpu.make_async_copy(src_hbm.at[pl.ds(base + ci * K, K)],
                              src_v.at[b6], srcsem.at[b6]).wait()
        pltpu.async_copy(m_hbm.at[src_v.at[b6]], rows_v.at[b3], gsem.at[b3])

    def g_wait(b6, b3):
        pltpu.make_async_copy(m_hbm.at[src_v.at[b6]], rows_v.at[b3],
                              gsem.at[b3]).wait()

    def ie_wait(ci, b6):
        off = base + ci * K
        pltpu.make_async_copy(dst_hbm.at[pl.ds(off, K)], dst_v.at[b6],
                              iesem.at[b6]).wait()
        pltpu.make_async_copy(ea_hbm.at[pl.ds(off, K)],
                              ea_v.at[b6, pl.ds(0, K)], iesem.at[b6]).wait()

    def s_start(b3, b6):
        pltpu.async_copy(rows_v.at[b3], agg_sh.at[dst_v.at[b6]],
                         ssem.at[b3], add=True)

    def s_wait(b3, b6):
        pltpu.make_async_copy(rows_v.at[b3], agg_sh.at[dst_v.at[b6]],
                              ssem.at[b3]).wait()

    def scale(b3, b6):
        def grp(g, _):
            eag = ea_v[b6, pl.ds(g * 16, 16)]
            for l in range(16):
                eav = jnp.full((16,), eag[l], jnp.float32)
                e = g * 16 + l
                for j in range(H // 16):
                    sl = pl.ds(16 * j, 16)
                    rows_v[b3, e, sl] = rows_v[b3, e, sl] * eav
            return 0

        lax.fori_loop(0, K // 16, grp, 0)

    for ci in range(4):
        idx_fetch(ci, ci)
    g_start(0, 0, 0)
    pltpu.sync_copy(z_hbm, agg_sh.at[pl.ds(s * RPT, RPT)])
    plsc.subcore_barrier()

    def step(ci, b6, b3, fetch_ahead, gather_next):
        g_wait(b6, b3)
        ie_wait(ci, b6)
        if isinstance(ci, int) and ci < 2:
            pass
        else:
            s_wait((b3 + 1) % NRB, (b6 + 4) % NIB)
        if fetch_ahead:
            idx_fetch(ci + 4, (b6 + 4) % NIB)
        if gather_next:
            g_start(ci + 1, (b6 + 1) % NIB, (b3 + 1) % NRB)
        scale(b3, b6)
        s_start(b3, b6)

    for ci in range(PEEL):
        step(ci, ci % NIB, ci % NRB, True, True)

    def sextet(t, carry):
        for k in range(6):
            ci = PEEL + 6 * t + k
            step(ci, (PEEL + k) % NIB, (PEEL + k) % NRB, True, True)
        return carry

    lax.fori_loop(0, (NCH - PEEL) // 6 - 1, sextet, 0)

    for k in range(6):
        ci = NCH - 6 + k
        b6, b3 = ci % NIB, ci % NRB
        g_wait(b6, b3)
        ie_wait(ci, b6)
        s_wait((b3 + 1) % NRB, (b6 + 4) % NIB)
        if ci + 4 < NCH:
            idx_fetch(ci + 4, (b6 + 4) % NIB)
        if ci + 1 < NCH:
            g_start(ci + 1, (b6 + 1) % NIB, (b3 + 1) % NRB)
        scale(b3, b6)
        s_start(b3, b6)

    s_wait((NCH - 2) % NRB, (NCH - 2) % NIB)
    s_wait((NCH - 1) % NRB, (NCH - 1) % NIB)
    plsc.subcore_barrier()

    pltpu.sync_copy(agg_sh.at[pl.ds(s * RPT, RPT)],
                    out_hbm.at[c, pl.ds(s * RPT, RPT)])


@functools.partial(
    pl.kernel,
    out_type=jax.ShapeDtypeStruct((2, NP, H), jnp.float32),
    mesh=plsc.VectorSubcoreMesh(core_axis_name="c", subcore_axis_name="s"),
    scratch_types=[
        pltpu.VMEM((NIB, K), jnp.int32),
        pltpu.VMEM((NIB, K), jnp.int32),
        pltpu.VMEM((NIB, K + 16), jnp.float32),
        pltpu.VMEM((NRB, K, H), jnp.float32),
        pltpu.VMEM_SHARED((NP, H), jnp.float32),
        pltpu.SemaphoreType.DMA((NRB,)),
        pltpu.SemaphoreType.DMA((NIB,)),
        pltpu.SemaphoreType.DMA((NIB,)),
        pltpu.SemaphoreType.DMA((NRB,)),
    ],
)
def _sc_edge(m_hbm, src_hbm, dst_hbm, ea_hbm, z_hbm, out_hbm,
             src_v, dst_v, ea_v, rows_v, agg_sh, gsem, srcsem, iesem, ssem):
    _sc_edge_body(m_hbm, src_hbm, dst_hbm, ea_hbm, z_hbm, out_hbm,
                  src_v, dst_v, ea_v, rows_v, agg_sh,
                  gsem, srcsem, iesem, ssem)


BN = 1000


def _mm_body(x_ref, w_ref, o_ref):
    o_ref[...] = jnp.dot(x_ref[...], w_ref[...],
                         preferred_element_type=jnp.float32)


def _tc_matmul(x, w):
    return pl.pallas_call(
        _mm_body,
        grid=(N // BN,),
        in_specs=[pl.BlockSpec((BN, H), lambda i: (i, 0)),
                  pl.BlockSpec((H, H), lambda i: (0, 0))],
        out_specs=pl.BlockSpec((BN, H), lambda i: (i, 0)),
        out_shape=jax.ShapeDtypeStruct((N, H), jnp.float32),
    )(x, w)


def _gru_math(p0, p1, h, wihT, whhT, bih, bhh):
    agg = p0 + p1
    gi = jnp.dot(agg, wihT, preferred_element_type=jnp.float32) + bih
    gh = jnp.dot(h, whhT, preferred_element_type=jnp.float32) + bhh
    r = jax.nn.sigmoid(gi[:, :H] + gh[:, :H])
    z = jax.nn.sigmoid(gi[:, H:2 * H] + gh[:, H:2 * H])
    n = jnp.tanh(gi[:, 2 * H:] + r * gh[:, 2 * H:])
    return (1.0 - z) * n + z * h


def _gru_fused_body(p0_ref, p1_ref, h_ref, wihT_ref, whhT_ref, bih_ref,
                    bhh_ref, wn_ref, oh_ref, om_ref):
    hn = _gru_math(p0_ref[0], p1_ref[0], h_ref[...], wihT_ref[...],
                   whhT_ref[...], bih_ref[...], bhh_ref[...])
    oh_ref[...] = hn
    om_ref[...] = jnp.dot(hn, wn_ref[...], preferred_element_type=jnp.float32)


def _gru_final_body(p0_ref, p1_ref, h_ref, wihT_ref, whhT_ref, bih_ref,
                    bhh_ref, oh_ref):
    oh_ref[...] = _gru_math(p0_ref[0], p1_ref[0], h_ref[...], wihT_ref[...],
                            whhT_ref[...], bih_ref[...], bhh_ref[...])


def _blk(r, c):
    return pl.BlockSpec((r, c), lambda i: (i, 0))


def _full(r, c):
    return pl.BlockSpec((r, c), lambda i: (0, 0))


def _pblk(core):
    return pl.BlockSpec((1, BN, H), lambda i, core=core: (core, i, 0))


def _tc_gru_fused(p, h, wihT, whhT, bih, bhh, wn):
    return pl.pallas_call(
        _gru_fused_body,
        grid=(N // BN,),
        in_specs=[_pblk(0), _pblk(1), _blk(BN, H),
                  _full(H, 3 * H), _full(H, 3 * H),
                  _full(1, 3 * H), _full(1, 3 * H), _full(H, H)],
        out_specs=[_blk(BN, H), _blk(BN, H)],
        out_shape=[jax.ShapeDtypeStruct((N, H), jnp.float32),
                   jax.ShapeDtypeStruct((N, H), jnp.float32)],
    )(p, p, h, wihT, whhT, bih, bhh, wn)


def _tc_gru_final(p, h, wihT, whhT, bih, bhh):
    return pl.pallas_call(
        _gru_final_body,
        grid=(N // BN,),
        in_specs=[_pblk(0), _pblk(1), _blk(BN, H),
                  _full(H, 3 * H), _full(H, 3 * H),
                  _full(1, 3 * H), _full(1, 3 * H)],
        out_specs=_blk(BN, H),
        out_shape=jax.ShapeDtypeStruct((N, H), jnp.float32),
    )(p, p, h, wihT, whhT, bih, bhh)


def kernel(x, edge_index, edge_attr, weight, w_ih, w_hh, b_ih, b_hh):
    src = edge_index[0].astype(jnp.int32)
    dst = edge_index[1].astype(jnp.int32)
    ea = edge_attr.astype(jnp.float32)
    wihT = w_ih.T
    whhT = w_hh.T
    bih = b_ih.reshape(1, 3 * H)
    bhh = b_hh.reshape(1, 3 * H)
    zeros = jnp.zeros((RPT, H), jnp.float32)

    h = x
    m = _tc_matmul(h, weight[0])
    p = _sc_edge(m, src, dst, ea, zeros)
    h, m = _tc_gru_fused(p, h, wihT, whhT, bih, bhh, weight[1])
    p = _sc_edge(m, src, dst, ea, zeros)
    h = _tc_gru_final(p, h, wihT, whhT, bih, bhh)
    return h
